# baseline (speedup 1.0000x reference)
import jax
import jax.numpy as jnp
from jax.experimental import pallas as pl
from jax.experimental.pallas import tpu as pltpu

D = 128
HEAD = 8
HEAD_DIM = 16
KATT = 7
KCONV = 3
KK = KCONV * KCONV
B = 16
L = 1024
NB = 2
NSTEP = B // NB


def _rshift(a, d):
    if d == 0:
        return a
    nseg = a.shape[1] // L
    pieces = []
    for j in range(nseg):
        o = j * L
        if d < 0:
            pieces += [a[:, o - d - l:o - d - l + 1] for l in range(-d)]
            pieces.append(a[:, o:o + L + d])
        else:
            pieces.append(a[:, o + d:o + L])
            pieces += [a[:, o + 2 * (L - 1) - (l + d):
                         o + 2 * (L - 1) - (l + d) + 1]
                       for l in range(L - d, L)]
    return jnp.concatenate(pieces, axis=1)


def _zshift(a, d):
    nseg = a.shape[1] // L
    z = jnp.zeros((a.shape[0], abs(d)), a.dtype)
    pieces = []
    for j in range(nseg):
        o = j * L
        if d < 0:
            pieces += [z, a[:, o:o + L + d]]
        else:
            pieces += [a[:, o + d:o + L], z]
    return jnp.concatenate(pieces, axis=1)


def _acmix_core(big, bq, pediff, bfield):
    f32 = jnp.float32
    qs = big[0:D] + bq
    k = big[D:2 * D]
    v = big[2 * D:3 * D]
    out_conv = (_zshift(big[3 * D:4 * D], -1) + big[4 * D:5 * D]
                + _zshift(big[5 * D:6 * D], 1))

    hh = jax.lax.broadcasted_iota(jnp.int32, (HEAD, D), 0)
    hc = jax.lax.broadcasted_iota(jnp.int32, (HEAD, D), 1)
    hsum = (hc // HEAD_DIM == hh).astype(f32)
    gh = jax.lax.broadcasted_iota(jnp.int32, (D, HEAD), 1)
    gc = jax.lax.broadcasted_iota(jnp.int32, (D, HEAD), 0)
    hrep = (gc // HEAD_DIM == gh).astype(f32)

    atts = []
    for t in range(KATT):
        terms = qs * (_rshift(k, t - 3) + pediff[t])
        atts.append(jnp.dot(hsum, terms, preferred_element_type=f32))
    m = atts[0]
    for a in atts[1:]:
        m = jnp.maximum(m, a)
    es = [jnp.exp(a - m) for a in atts]
    den = es[0]
    for e in es[1:]:
        den = den + e
    inv = 1.0 / den
    out_att = jnp.zeros((D, NB * L), f32)
    for t in range(KATT):
        wfull = jnp.dot(hrep, es[t] * inv, preferred_element_type=f32)
        out_att = out_att + wfull * _rshift(v, t - 3)

    return jnp.maximum(out_att + out_conv + bfield, 0.0)


def _embed_kernel(v_ref, embT_ref, x_ref):
    f32 = jnp.float32
    iota = jax.lax.broadcasted_iota(jnp.int32, (32, L), 0)
    oh = jnp.concatenate(
        [(iota == jnp.clip(v_ref[j], 0, 25)).astype(f32) for j in range(NB)],
        axis=1)
    x = jnp.dot(embT_ref[...], oh, preferred_element_type=f32)
    for j in range(NB):
        x_ref[j] = x[:, j * L:(j + 1) * L]


def _stat_fold(acc, gb):
    n = float(B * L)
    mean = acc[:, 0:1] / n
    var = acc[:, 1:2] / n - mean * mean
    s = gb[:, 0:1] * jax.lax.rsqrt(var + 1e-5)
    return s, gb[:, 1:2] - mean * s


def _layers_kernel(x_ref, wall_ref, bq_ref, bngb_ref, pediff_ref, bf_ref,
                   xo_ref, accout_ref, acc_ref, st_ref):
    f32 = jnp.float32
    li = pl.program_id(0)
    bi = pl.program_id(1)

    @pl.when(bi == 0)
    def _boundary():
        @pl.when(li == 0)
        def _first():
            st_ref[...] = jnp.concatenate(
                [jnp.ones((D, 1), f32), jnp.zeros((D, 7), f32)], axis=1)

        @pl.when(li > 0)
        def _fold():
            s, t = _stat_fold(acc_ref[...], bngb_ref[0])
            st_ref[...] = jnp.concatenate(
                [s, t, jnp.zeros((D, 6), f32)], axis=1)

        acc_ref[...] = jnp.zeros((D, 8), f32)

    st = st_ref[...]
    xx = jnp.concatenate([x_ref[j] for j in range(NB)], axis=1)
    xn = xx * st[:, 0:1] + st[:, 1:2]
    big = jnp.dot(wall_ref[0], xn, preferred_element_type=f32)
    y = _acmix_core(big, bq_ref[0], pediff_ref[0], bf_ref[0])
    for j in range(NB):
        xo_ref[j] = y[:, j * L:(j + 1) * L]
    ssum = jnp.sum(y, axis=1, keepdims=True)
    ssq = jnp.sum(y * y, axis=1, keepdims=True)
    acc_ref[...] = acc_ref[...] + jnp.concatenate(
        [ssum, ssq, jnp.zeros((D, 6), f32)], axis=1)

    @pl.when(jnp.logical_and(li == 2, bi == NSTEP - 1))
    def _export():
        accout_ref[...] = acc_ref[...]


def _norm_kernel(x_ref, acc_ref, gb_ref, o_ref):
    s, t = _stat_fold(acc_ref[...], gb_ref[...])
    for j in range(NB):
        o_ref[j] = x_ref[j] * s + t


def _const_spec(shape):
    n = len(shape)
    return pl.BlockSpec(shape, lambda *a: (0,) * n)


def _layer_weights(lp):
    f32 = jnp.float32
    wq, bq = lp['conv1_w'], lp['conv1_b']
    wk, bk = lp['conv2_w'], lp['conv2_b']
    wv, bv = lp['conv3_w'], lp['conv3_b']
    t9 = lp['fc_w'].reshape(KK, 3, HEAD)
    wf = jnp.einsum('cph,de->cdphe', t9, jnp.eye(HEAD_DIM, dtype=f32))
    wf = wf.reshape(KK * HEAD_DIM, 3 * D)
    oh_g = (jnp.arange(HEAD_DIM)[None, :]
            == (jnp.arange(D) // (D // HEAD_DIM))[:, None]).astype(f32)
    wd3 = jnp.einsum('xct,xg->txcg', lp['dep_w'], oh_g) * lp['rate2']
    wd3 = wd3.reshape(KCONV * D, KK * HEAD_DIM)
    wqkv = jnp.concatenate([wq, wk, wv], axis=0)
    bqkv = jnp.concatenate([bq, bk, bv], axis=0)[:, None]
    hi = jax.lax.Precision.HIGHEST
    wc = jnp.dot(wd3, jnp.dot(wf, wqkv, precision=hi),
                 precision=hi)
    bc = jnp.dot(wd3, jnp.dot(wf, bqkv, precision=hi),
                 precision=hi)
    bc = bc.at[D:2 * D, 0].add(lp['rate2'] * lp['dep_b'])
    scale = float(HEAD_DIM) ** -0.5
    w_all = jnp.concatenate(
        [wq * scale, wk, wv * lp['rate1'], wc], axis=0)
    loc = jnp.stack([jnp.linspace(-1.0, 1.0, L),
                     -jnp.ones((L,), dtype=f32)], axis=0)
    pe = jnp.dot(lp['conv_p_w'], loc) + lp['conv_p_b'][:, None]
    pep = jnp.pad(pe, ((0, 0), (3, 3)), mode='reflect')
    pediff = jnp.stack([pe - pep[:, t:t + L] for t in range(KATT)], axis=0)
    pediff = jnp.tile(pediff, (1, HEAD, NB)) + bk[None, :, None]
    li = jnp.arange(L)
    bf = (bc[D:2 * D, 0:1] + lp['rate1'] * bv[:, None]
          + jnp.where(li[None, :] > 0, bc[0:D, 0:1], 0.0)
          + jnp.where(li[None, :] < L - 1, bc[2 * D:3 * D, 0:1], 0.0))
    bf = jnp.tile(bf, (1, NB))
    return w_all, bq[:, None] * scale, pediff, bf


def kernel(v, params):
    f32 = jnp.float32
    v3 = v.astype(jnp.int32).reshape(B, 1, L)
    lps = [params['layer%d' % i] for i in range(3)]

    embT = jnp.zeros((D, 32), f32).at[:, :26].set(params['emb'].T)
    x0 = pl.pallas_call(
        _embed_kernel,
        grid=(NSTEP,),
        in_specs=[pl.BlockSpec((NB, 1, L), lambda b: (b, 0, 0)),
                  _const_spec((D, 32))],
        out_specs=pl.BlockSpec((NB, D, L), lambda b: (b, 0, 0)),
        out_shape=jax.ShapeDtypeStruct((B, D, L), f32),
    )(v3, embT)

    prep = [_layer_weights(lp) for lp in lps]
    wall3 = jnp.stack([p[0] for p in prep])
    bq3 = jnp.stack([p[1] for p in prep])
    pediff3 = jnp.stack([p[2] for p in prep])
    bf3 = jnp.stack([p[3] for p in prep])
    bngb3 = jnp.stack(
        [jnp.stack([lp['bn_g'], lp['bn_b']] + [jnp.zeros((D,), f32)] * 6,
                   axis=1) for lp in lps])

    x, acc = pl.pallas_call(
        _layers_kernel,
        grid=(3, NSTEP),
        in_specs=[
            pl.BlockSpec((NB, D, L), lambda l, b: (b, 0, 0)),
            pl.BlockSpec((1, 6 * D, D), lambda l, b: (l, 0, 0)),
            pl.BlockSpec((1, D, 1), lambda l, b: (l, 0, 0)),
            pl.BlockSpec((1, D, 8),
                         lambda l, b: (jnp.maximum(l - 1, 0), 0, 0)),
            pl.BlockSpec((1, KATT, D, NB * L), lambda l, b: (l, 0, 0, 0)),
            pl.BlockSpec((1, D, NB * L), lambda l, b: (l, 0, 0)),
        ],
        out_specs=[pl.BlockSpec((NB, D, L), lambda l, b: (b, 0, 0)),
                   _const_spec((D, 8))],
        out_shape=[jax.ShapeDtypeStruct((B, D, L), f32),
                   jax.ShapeDtypeStruct((D, 8), f32)],
        scratch_shapes=[pltpu.VMEM((D, 8), f32), pltpu.VMEM((D, 8), f32)],
        input_output_aliases={0: 0},
    )(x0, wall3, bq3, bngb3, pediff3, bf3)

    y = pl.pallas_call(
        _norm_kernel,
        grid=(NSTEP,),
        in_specs=[pl.BlockSpec((NB, D, L), lambda b: (b, 0, 0)),
                  _const_spec((D, 8)),
                  _const_spec((D, 8))],
        out_specs=pl.BlockSpec((NB, D, L), lambda b: (b, 0, 0)),
        out_shape=jax.ShapeDtypeStruct((B, D, L), f32),
    )(x, acc, bngb3[2])
    return y.reshape(B, L, D)


# speedup vs baseline: 1.5136x; 1.5136x over previous
import jax
import jax.numpy as jnp
from jax.experimental import pallas as pl
from jax.experimental.pallas import tpu as pltpu

D = 128
HEAD = 8
HEAD_DIM = 16
KATT = 7
KCONV = 3
KK = KCONV * KCONV
B = 16
L = 1024
NB = 2
NSTEP = B // NB


def _rshift(a, d):
    if d == 0:
        return a
    nseg = a.shape[1] // L
    pieces = []
    for j in range(nseg):
        o = j * L
        if d < 0:
            pieces += [a[:, o - d - l:o - d - l + 1] for l in range(-d)]
            pieces.append(a[:, o:o + L + d])
        else:
            pieces.append(a[:, o + d:o + L])
            pieces += [a[:, o + 2 * (L - 1) - (l + d):
                         o + 2 * (L - 1) - (l + d) + 1]
                       for l in range(L - d, L)]
    return jnp.concatenate(pieces, axis=1)


def _zshift(a, d):
    nseg = a.shape[1] // L
    z = jnp.zeros((a.shape[0], abs(d)), a.dtype)
    pieces = []
    for j in range(nseg):
        o = j * L
        if d < 0:
            pieces += [z, a[:, o:o + L + d]]
        else:
            pieces += [a[:, o + d:o + L], z]
    return jnp.concatenate(pieces, axis=1)


def _acmix_core(big, ball, pd_ref):
    f32 = jnp.float32
    big = big + ball
    qs = big[0:D]
    k = big[D:2 * D]
    v = big[2 * D:3 * D]
    out_conv = (_zshift(big[3 * D:4 * D], -1) + big[4 * D:5 * D]
                + _zshift(big[5 * D:6 * D], 1))

    hh = jax.lax.broadcasted_iota(jnp.int32, (HEAD, D), 0)
    hc = jax.lax.broadcasted_iota(jnp.int32, (HEAD, D), 1)
    hsum = (hc // HEAD_DIM == hh).astype(f32)
    gh = jax.lax.broadcasted_iota(jnp.int32, (D, HEAD), 1)
    gc = jax.lax.broadcasted_iota(jnp.int32, (D, HEAD), 0)
    hrep = (gc // HEAD_DIM == gh).astype(f32)

    atts = []
    for t in range(KATT):
        terms = qs * (_rshift(k, t - 3) + pd_ref[t])
        atts.append(jnp.dot(hsum, terms, preferred_element_type=f32))
    m = atts[0]
    for a in atts[1:]:
        m = jnp.maximum(m, a)
    es = [jnp.exp(a - m) for a in atts]
    den = es[0]
    for e in es[1:]:
        den = den + e
    inv = 1.0 / den
    out_att = jnp.zeros((D, NB * L), f32)
    for t in range(KATT):
        wfull = jnp.dot(hrep, es[t] * inv, preferred_element_type=f32)
        out_att = out_att + wfull * _rshift(v, t - 3)

    return jnp.maximum(out_att + out_conv, 0.0)


def _embed_kernel(v_ref, embT_ref, x_ref):
    f32 = jnp.float32
    iota = jax.lax.broadcasted_iota(jnp.int32, (32, L), 0)
    oh = jnp.concatenate(
        [(iota == jnp.clip(v_ref[j], 0, 25)).astype(f32) for j in range(NB)],
        axis=1)
    x = jnp.dot(embT_ref[...], oh, preferred_element_type=f32)
    for j in range(NB):
        x_ref[j] = x[:, j * L:(j + 1) * L]


def _stat_fold(acc, gb):
    n = float(B * L)
    mean = acc[:, 0:1] / n
    var = acc[:, 1:2] / n - mean * mean
    s = gb[:, 0:1] * jax.lax.rsqrt(var + 1e-5)
    return s, gb[:, 1:2] - mean * s


def _layers_kernel(x_ref, wall_ref, ball_ref, bngb_ref, pds_ref,
                   xo_ref, accout_ref, acc_ref, st_ref, pd_ref):
    f32 = jnp.float32
    li = pl.program_id(0)
    bi = pl.program_id(1)

    @pl.when(bi == 0)
    def _boundary():
        @pl.when(li == 0)
        def _first():
            st_ref[...] = jnp.concatenate(
                [jnp.ones((D, 1), f32), jnp.zeros((D, 7), f32)], axis=1)

        @pl.when(li > 0)
        def _fold():
            s, t = _stat_fold(acc_ref[...], bngb_ref[0])
            st_ref[...] = jnp.concatenate(
                [s, t, jnp.zeros((D, 6), f32)], axis=1)

        acc_ref[...] = jnp.zeros((D, 8), f32)
        tr = jax.lax.broadcasted_iota(jnp.int32, (D, HEAD_DIM), 0)
        tc = jax.lax.broadcasted_iota(jnp.int32, (D, HEAD_DIM), 1)
        tile = (tr % HEAD_DIM == tc).astype(f32)
        for t in range(KATT):
            pdt = jnp.dot(tile, pds_ref[0][t], preferred_element_type=f32)
            pd_ref[t] = jnp.concatenate([pdt] * NB, axis=1)

    st = st_ref[...]
    xx = jnp.concatenate([x_ref[j] for j in range(NB)], axis=1)
    xn = xx * st[:, 0:1] + st[:, 1:2]
    big = jnp.dot(wall_ref[0], xn, preferred_element_type=f32)
    y = _acmix_core(big, ball_ref[0], pd_ref)
    for j in range(NB):
        xo_ref[j] = y[:, j * L:(j + 1) * L]
    ssum = jnp.sum(y, axis=1, keepdims=True)
    ssq = jnp.sum(y * y, axis=1, keepdims=True)
    acc_ref[...] = acc_ref[...] + jnp.concatenate(
        [ssum, ssq, jnp.zeros((D, 6), f32)], axis=1)

    @pl.when(jnp.logical_and(li == 2, bi == NSTEP - 1))
    def _export():
        accout_ref[...] = acc_ref[...]


def _norm_kernel(x_ref, acc_ref, gb_ref, o_ref):
    s, t = _stat_fold(acc_ref[...], gb_ref[...])
    for j in range(NB):
        o_ref[j] = x_ref[j] * s + t


def _const_spec(shape):
    n = len(shape)
    return pl.BlockSpec(shape, lambda *a: (0,) * n)


def _layer_weights(lp):
    f32 = jnp.float32
    wq, bq = lp['conv1_w'], lp['conv1_b']
    wk, bk = lp['conv2_w'], lp['conv2_b']
    wv, bv = lp['conv3_w'], lp['conv3_b']
    t9 = lp['fc_w'].reshape(KK, 3, HEAD)
    wf = jnp.einsum('cph,de->cdphe', t9, jnp.eye(HEAD_DIM, dtype=f32))
    wf = wf.reshape(KK * HEAD_DIM, 3 * D)
    oh_g = (jnp.arange(HEAD_DIM)[None, :]
            == (jnp.arange(D) // (D // HEAD_DIM))[:, None]).astype(f32)
    wd3 = jnp.einsum('xct,xg->txcg', lp['dep_w'], oh_g) * lp['rate2']
    wd3 = wd3.reshape(KCONV * D, KK * HEAD_DIM)
    wqkv = jnp.concatenate([wq, wk, wv], axis=0)
    bqkv = jnp.concatenate([bq, bk, bv], axis=0)[:, None]
    hi = jax.lax.Precision.HIGHEST
    wc = jnp.dot(wd3, jnp.dot(wf, wqkv, precision=hi),
                 precision=hi)
    bc = jnp.dot(wd3, jnp.dot(wf, bqkv, precision=hi),
                 precision=hi)
    bc = bc.at[D:2 * D, 0].add(lp['rate2'] * lp['dep_b'])
    scale = float(HEAD_DIM) ** -0.5
    w_all = jnp.concatenate(
        [wq * scale, wk, wv * lp['rate1'], wc], axis=0)
    b_all = jnp.concatenate(
        [bq[:, None] * scale, bk[:, None], bv[:, None] * lp['rate1'], bc],
        axis=0)
    loc = jnp.stack([jnp.linspace(-1.0, 1.0, L),
                     -jnp.ones((L,), dtype=f32)], axis=0)
    pe = jnp.dot(lp['conv_p_w'], loc) + lp['conv_p_b'][:, None]
    pep = jnp.pad(pe, ((0, 0), (3, 3)), mode='reflect')
    pediff = jnp.stack([pe - pep[:, t:t + L] for t in range(KATT)], axis=0)
    return w_all, b_all, pediff


def kernel(v, params):
    f32 = jnp.float32
    v3 = v.astype(jnp.int32).reshape(B, 1, L)
    lps = [params['layer%d' % i] for i in range(3)]

    embT = jnp.zeros((D, 32), f32).at[:, :26].set(params['emb'].T)
    x0 = pl.pallas_call(
        _embed_kernel,
        grid=(NSTEP,),
        in_specs=[pl.BlockSpec((NB, 1, L), lambda b: (b, 0, 0)),
                  _const_spec((D, 32))],
        out_specs=pl.BlockSpec((NB, D, L), lambda b: (b, 0, 0)),
        out_shape=jax.ShapeDtypeStruct((B, D, L), f32),
    )(v3, embT)

    prep = [_layer_weights(lp) for lp in lps]
    wall3 = jnp.stack([p[0] for p in prep])
    ball3 = jnp.stack([p[1] for p in prep])
    pds3 = jnp.stack([p[2] for p in prep])
    bngb3 = jnp.stack(
        [jnp.stack([lp['bn_g'], lp['bn_b']] + [jnp.zeros((D,), f32)] * 6,
                   axis=1) for lp in lps])

    x, acc = pl.pallas_call(
        _layers_kernel,
        grid=(3, NSTEP),
        in_specs=[
            pl.BlockSpec((NB, D, L), lambda l, b: (b, 0, 0)),
            pl.BlockSpec((1, 6 * D, D), lambda l, b: (l, 0, 0)),
            pl.BlockSpec((1, 6 * D, 1), lambda l, b: (l, 0, 0)),
            pl.BlockSpec((1, D, 8),
                         lambda l, b: (jnp.maximum(l - 1, 0), 0, 0)),
            pl.BlockSpec((1, KATT, HEAD_DIM, L), lambda l, b: (l, 0, 0, 0)),
        ],
        out_specs=[pl.BlockSpec((NB, D, L), lambda l, b: (b, 0, 0)),
                   _const_spec((D, 8))],
        out_shape=[jax.ShapeDtypeStruct((B, D, L), f32),
                   jax.ShapeDtypeStruct((D, 8), f32)],
        scratch_shapes=[pltpu.VMEM((D, 8), f32), pltpu.VMEM((D, 8), f32),
                        pltpu.VMEM((KATT, D, NB * L), f32)],
        input_output_aliases={0: 0},
    )(x0, wall3, ball3, bngb3, pds3)

    y = pl.pallas_call(
        _norm_kernel,
        grid=(NSTEP,),
        in_specs=[pl.BlockSpec((NB, D, L), lambda b: (b, 0, 0)),
                  _const_spec((D, 8)),
                  _const_spec((D, 8))],
        out_specs=pl.BlockSpec((NB, D, L), lambda b: (b, 0, 0)),
        out_shape=jax.ShapeDtypeStruct((B, D, L), f32),
    )(x, acc, bngb3[2])
    return y.reshape(B, L, D)
